# baseline (speedup 1.0000x reference)
import functools

import jax
import jax.numpy as jnp
from jax import lax
from jax.experimental import pallas as pl
from jax.experimental.pallas import tpu as pltpu
from jax.experimental.pallas import tpu_sc as plsc

N = 10000
E = 320000
NFEAT = 128
NHID = 128
NCLASS = 64

NC = 2
NS = 16
NW = NC * NS
EPW = E // NW
CHUNK = 80
NCHUNKS = EPW // CHUNK
ROWS_MAIN = 640
ROWS_LAST = N - (NS - 1) * ROWS_MAIN
RB = 80


def _make_spmm(F, from_spmem=False, feature_split=False):
    mesh = plsc.VectorSubcoreMesh(
        core_axis_name="c", subcore_axis_name="s", num_cores=NC, num_subcores=NS)

    nchunks = (E // CHUNK) // NS if feature_split else NCHUNKS

    scratch = [
        pltpu.VMEM((nchunks, CHUNK), jnp.int32),
        pltpu.VMEM((nchunks, CHUNK), jnp.int32),
        pltpu.VMEM((CHUNK, F), jnp.float32),
        pltpu.VMEM((CHUNK, F), jnp.float32),
        pltpu.VMEM((RB, F), jnp.float32),
        pltpu.VMEM_SHARED((N, F), jnp.float32),
        pltpu.SemaphoreType.DMA,
        pltpu.SemaphoreType.DMA,
        pltpu.SemaphoreType.DMA,
        pltpu.SemaphoreType.DMA,
    ]
    if from_spmem:
        scratch.append(pltpu.VMEM_SHARED((N, F), jnp.float32))

    @functools.partial(
        pl.kernel,
        out_type=jax.ShapeDtypeStruct((NC, N, F), jnp.float32),
        mesh=mesh,
        scratch_types=scratch,
        compiler_params=pltpu.CompilerParams(use_tc_tiling_on_sc=False),
    )
    def spmm(*args):
        if feature_split:
            (sup_a_hbm, sup_b_hbm, src_hbm, dst_hbm, out_hbm,
             src_v, dst_v, buf0, buf1, zbuf_v, acc_s,
             gs0, gs1, ss0, ss1, sup_s) = args
            support_hbm = None
        else:
            (support_hbm, src_hbm, dst_hbm, out_hbm,
             src_v, dst_v, buf0, buf1, zbuf_v, acc_s,
             gs0, gs1, ss0, ss1, *maybe_sup) = args
            sup_s = maybe_sup[0] if from_spmem else None
        bufs = (buf0, buf1)
        gsems = (gs0, gs1)
        ssems = (ss0, ss1)
        cid = lax.axis_index("c")
        sid = lax.axis_index("s")
        wid = cid * NS + sid

        zvec = jnp.zeros((16,), jnp.float32)

        def zrow(i, carry):
            for j in range(F // 16):
                zbuf_v[i, pl.ds(j * 16, 16)] = zvec
            return carry
        lax.fori_loop(0, RB, zrow, 0)

        row0 = sid * ROWS_MAIN
        nz = jnp.where(sid == NS - 1, ROWS_LAST // RB, ROWS_MAIN // RB)

        def zchunk(k, carry):
            pltpu.sync_copy(zbuf_v, acc_s.at[pl.ds(row0 + k * RB, RB)])
            return carry
        lax.fori_loop(0, nz, zchunk, 0)

        slab = (sid if feature_split else wid) * nchunks
        pltpu.sync_copy(src_hbm.at[pl.ds(slab, nchunks)], src_v)
        pltpu.sync_copy(dst_hbm.at[pl.ds(slab, nchunks)], dst_v)

        if from_spmem:
            def schunk(k, carry):
                r = row0 + k * RB
                if feature_split:
                    @pl.when(cid == 0)
                    def _():
                        pltpu.sync_copy(sup_a_hbm.at[pl.ds(r, RB)],
                                        sup_s.at[pl.ds(r, RB)])

                    @pl.when(cid == 1)
                    def _():
                        pltpu.sync_copy(sup_b_hbm.at[pl.ds(r, RB)],
                                        sup_s.at[pl.ds(r, RB)])
                else:
                    pltpu.sync_copy(support_hbm.at[pl.ds(r, RB)],
                                    sup_s.at[pl.ds(r, RB)])
                return carry
            lax.fori_loop(0, nz, schunk, 0)
        gsrc = sup_s if from_spmem else support_hbm

        NB, LA = 2, 1

        def gstart(t, b):
            pltpu.async_copy(gsrc.at[src_v.at[t]], bufs[b], gsems[b])

        def gwait(t, b):
            pltpu.make_async_copy(
                gsrc.at[src_v.at[t]], bufs[b], gsems[b]).wait()

        def sstart(t, b):
            pltpu.async_copy(bufs[b], acc_s.at[dst_v.at[t]], ssems[b], add=True)

        def swait(t, b):
            pltpu.make_async_copy(
                bufs[b], acc_s.at[dst_v.at[t]], ssems[b]).wait()

        if not from_spmem:
            for t0 in range(LA):
                gstart(t0, t0)
        plsc.subcore_barrier()
        if from_spmem:
            for t0 in range(LA):
                gstart(t0, t0)

        def body(t, carry):
            for b in range(NB):
                @pl.when(lax.rem(t, NB) == b)
                def _():
                    gwait(t, b)
                    sstart(t, b)
                    b2 = (b + LA) % NB

                    @pl.when(t + LA < nchunks)
                    def _():
                        @pl.when(t >= NB - LA)
                        def _():
                            swait(t - (NB - LA), b2)
                        gstart(t + LA, b2)
            return carry
        lax.fori_loop(0, nchunks, body, 0)

        for tf in range(nchunks - NB, nchunks):
            swait(tf, tf % NB)

        plsc.subcore_barrier()

        def wchunk(k, carry):
            r = row0 + k * RB
            pltpu.sync_copy(acc_s.at[pl.ds(r, RB)], out_hbm.at[cid, pl.ds(r, RB)])
            return carry
        lax.fori_loop(0, nz, wchunk, 0)

    return spmm


_FH = NHID // 2
_spmm_h = _make_spmm(_FH, from_spmem=True)
_spmm_c = _make_spmm(NCLASS, from_spmem=True)


_BR = 1000


def _fuse_body(pa_ref, pb_ref, w1_ref, b_ref, w2_ref, o_ref):
    agg = jnp.concatenate(
        [pa_ref[0] + pa_ref[1], pb_ref[0] + pb_ref[1]], axis=-1)
    h = jnp.maximum(
        jnp.dot(agg, w1_ref[...], preferred_element_type=jnp.float32)
        + b_ref[...], 0.0)
    o_ref[...] = jnp.dot(h, w2_ref[...], preferred_element_type=jnp.float32)


def _fuse1(parts_a, parts_b, w1, b, w2):
    return pl.pallas_call(
        _fuse_body,
        grid=(N // _BR,),
        in_specs=[pl.BlockSpec((NC, _BR, _FH), lambda i: (0, i, 0)),
                  pl.BlockSpec((NC, _BR, _FH), lambda i: (0, i, 0)),
                  pl.BlockSpec((NFEAT, NHID), lambda i: (0, 0)),
                  pl.BlockSpec((1, NHID), lambda i: (0, 0)),
                  pl.BlockSpec((NHID, NCLASS), lambda i: (0, 0))],
        out_specs=pl.BlockSpec((_BR, NCLASS), lambda i: (i, 0)),
        out_shape=jax.ShapeDtypeStruct((N, NCLASS), jnp.float32),
    )(parts_a, parts_b, w1, b, w2)


def _lsm_body(p_ref, b_ref, o_ref):
    z = p_ref[0] + p_ref[1] + b_ref[...]
    m = jnp.max(z, axis=1, keepdims=True)
    e = jnp.exp(z - m)
    s = jnp.sum(e, axis=1, keepdims=True)
    o_ref[...] = z - m - jnp.log(s)


def _lsm(parts, b):
    return pl.pallas_call(
        _lsm_body,
        grid=(N // _BR,),
        in_specs=[pl.BlockSpec((NC, _BR, NCLASS), lambda i: (0, i, 0)),
                  pl.BlockSpec((1, NCLASS), lambda i: (0, 0))],
        out_specs=pl.BlockSpec((_BR, NCLASS), lambda i: (i, 0)),
        out_shape=jax.ShapeDtypeStruct((N, NCLASS), jnp.float32),
    )(parts, b)


def kernel(raw_x, edge_index, W1, b1, W2, b2):
    src = edge_index[0].reshape(E // CHUNK, CHUNK)
    dst = edge_index[1].reshape(E // CHUNK, CHUNK)
    parts_a = _spmm_h(raw_x[:, :_FH], src, dst)
    parts_b = _spmm_h(raw_x[:, _FH:], src, dst)
    support2 = _fuse1(parts_a, parts_b, W1, b1.reshape(1, NHID), W2)
    parts2 = _spmm_c(support2, src, dst)
    return _lsm(parts2, b2.reshape(1, NCLASS))


# speedup vs baseline: 1.1138x; 1.1138x over previous
import functools

import jax
import jax.numpy as jnp
from jax import lax
from jax.experimental import pallas as pl
from jax.experimental.pallas import tpu as pltpu
from jax.experimental.pallas import tpu_sc as plsc

N = 10000
E = 320000
NFEAT = 128
NHID = 128
NCLASS = 64

NC = 2
NS = 16
NW = NC * NS
EPW = E // NW
CHUNK = 80
NCHUNKS = EPW // CHUNK
ROWS_MAIN = 640
ROWS_LAST = N - (NS - 1) * ROWS_MAIN
RB = 80


def _make_spmm(F, from_spmem=False, feature_split=False):
    mesh = plsc.VectorSubcoreMesh(
        core_axis_name="c", subcore_axis_name="s", num_cores=NC, num_subcores=NS)

    nchunks = (E // CHUNK) // NS if feature_split else NCHUNKS

    scratch = [
        pltpu.VMEM((nchunks, CHUNK), jnp.int32),
        pltpu.VMEM((nchunks, CHUNK), jnp.int32),
        pltpu.VMEM((CHUNK, F), jnp.float32),
        pltpu.VMEM((CHUNK, F), jnp.float32),
        pltpu.VMEM((RB, F), jnp.float32),
        pltpu.VMEM_SHARED((N, F), jnp.float32),
        pltpu.SemaphoreType.DMA,
        pltpu.SemaphoreType.DMA,
        pltpu.SemaphoreType.DMA,
        pltpu.SemaphoreType.DMA,
    ]
    if from_spmem:
        scratch.append(pltpu.VMEM_SHARED((N, F), jnp.float32))

    @functools.partial(
        pl.kernel,
        out_type=jax.ShapeDtypeStruct((NC, N, F), jnp.float32),
        mesh=mesh,
        scratch_types=scratch,
        compiler_params=pltpu.CompilerParams(use_tc_tiling_on_sc=False),
    )
    def spmm(*args):
        if feature_split:
            (sup_a_hbm, sup_b_hbm, src_hbm, dst_hbm, out_hbm,
             src_v, dst_v, buf0, buf1, zbuf_v, acc_s,
             gs0, gs1, ss0, ss1, sup_s) = args
            support_hbm = None
        else:
            (support_hbm, src_hbm, dst_hbm, out_hbm,
             src_v, dst_v, buf0, buf1, zbuf_v, acc_s,
             gs0, gs1, ss0, ss1, *maybe_sup) = args
            sup_s = maybe_sup[0] if from_spmem else None
        bufs = (buf0, buf1)
        gsems = (gs0, gs1)
        ssems = (ss0, ss1)
        cid = lax.axis_index("c")
        sid = lax.axis_index("s")
        wid = cid * NS + sid

        zvec = jnp.zeros((16,), jnp.float32)

        def zrow(i, carry):
            for j in range(F // 16):
                zbuf_v[i, pl.ds(j * 16, 16)] = zvec
            return carry
        lax.fori_loop(0, RB, zrow, 0)

        row0 = sid * ROWS_MAIN
        nz = jnp.where(sid == NS - 1, ROWS_LAST // RB, ROWS_MAIN // RB)

        def zchunk(k, carry):
            pltpu.sync_copy(zbuf_v, acc_s.at[pl.ds(row0 + k * RB, RB)])
            return carry
        lax.fori_loop(0, nz, zchunk, 0)

        slab = (sid if feature_split else wid) * nchunks
        pltpu.sync_copy(src_hbm.at[pl.ds(slab, nchunks)], src_v)
        pltpu.sync_copy(dst_hbm.at[pl.ds(slab, nchunks)], dst_v)

        if from_spmem:
            def schunk(k, carry):
                r = row0 + k * RB
                if feature_split:
                    @pl.when(cid == 0)
                    def _():
                        pltpu.sync_copy(sup_a_hbm.at[pl.ds(r, RB)],
                                        sup_s.at[pl.ds(r, RB)])

                    @pl.when(cid == 1)
                    def _():
                        pltpu.sync_copy(sup_b_hbm.at[pl.ds(r, RB)],
                                        sup_s.at[pl.ds(r, RB)])
                else:
                    pltpu.sync_copy(support_hbm.at[pl.ds(r, RB)],
                                    sup_s.at[pl.ds(r, RB)])
                return carry
            lax.fori_loop(0, nz, schunk, 0)
        gsrc = sup_s if from_spmem else support_hbm

        NB, LA = 2, 1

        def gstart(t, b):
            pltpu.async_copy(gsrc.at[src_v.at[t]], bufs[b], gsems[b])

        def gwait(t, b):
            pltpu.make_async_copy(
                gsrc.at[src_v.at[t]], bufs[b], gsems[b]).wait()

        def sstart(t, b):
            pltpu.async_copy(bufs[b], acc_s.at[dst_v.at[t]], ssems[b], add=True)

        def swait(t, b):
            pltpu.make_async_copy(
                bufs[b], acc_s.at[dst_v.at[t]], ssems[b]).wait()

        if not from_spmem:
            for t0 in range(LA):
                gstart(t0, t0)
        plsc.subcore_barrier()
        if from_spmem:
            for t0 in range(LA):
                gstart(t0, t0)

        def body(t, carry):
            for b in range(NB):
                @pl.when(lax.rem(t, NB) == b)
                def _():
                    gwait(t, b)
                    sstart(t, b)
                    b2 = (b + LA) % NB

                    @pl.when(t + LA < nchunks)
                    def _():
                        @pl.when(t >= NB - LA)
                        def _():
                            swait(t - (NB - LA), b2)
                        gstart(t + LA, b2)
            return carry
        lax.fori_loop(0, nchunks, body, 0)

        for tf in range(nchunks - NB, nchunks):
            swait(tf, tf % NB)

        plsc.subcore_barrier()

        def wchunk(k, carry):
            r = row0 + k * RB
            pltpu.sync_copy(acc_s.at[pl.ds(r, RB)], out_hbm.at[cid, pl.ds(r, RB)])
            return carry
        lax.fori_loop(0, nz, wchunk, 0)

    return spmm


_spmm_h = _make_spmm(NHID)
_spmm_c = _make_spmm(NCLASS, from_spmem=True)


_BR = 1000


def _fuse_body(p_ref, w1_ref, b_ref, w2_ref, o_ref):
    agg = p_ref[0] + p_ref[1]
    h = jnp.maximum(
        jnp.dot(agg, w1_ref[...], preferred_element_type=jnp.float32)
        + b_ref[...], 0.0)
    o_ref[...] = jnp.dot(h, w2_ref[...], preferred_element_type=jnp.float32)


def _fuse1(parts, w1, b, w2):
    return pl.pallas_call(
        _fuse_body,
        grid=(N // _BR,),
        in_specs=[pl.BlockSpec((NC, _BR, NFEAT), lambda i: (0, i, 0)),
                  pl.BlockSpec((NFEAT, NHID), lambda i: (0, 0)),
                  pl.BlockSpec((1, NHID), lambda i: (0, 0)),
                  pl.BlockSpec((NHID, NCLASS), lambda i: (0, 0))],
        out_specs=pl.BlockSpec((_BR, NCLASS), lambda i: (i, 0)),
        out_shape=jax.ShapeDtypeStruct((N, NCLASS), jnp.float32),
    )(parts, w1, b, w2)


def _lsm_body(p_ref, b_ref, o_ref):
    z = p_ref[0] + p_ref[1] + b_ref[...]
    m = jnp.max(z, axis=1, keepdims=True)
    e = jnp.exp(z - m)
    s = jnp.sum(e, axis=1, keepdims=True)
    o_ref[...] = z - m - jnp.log(s)


def _lsm(parts, b):
    return pl.pallas_call(
        _lsm_body,
        grid=(N // _BR,),
        in_specs=[pl.BlockSpec((NC, _BR, NCLASS), lambda i: (0, i, 0)),
                  pl.BlockSpec((1, NCLASS), lambda i: (0, 0))],
        out_specs=pl.BlockSpec((_BR, NCLASS), lambda i: (i, 0)),
        out_shape=jax.ShapeDtypeStruct((N, NCLASS), jnp.float32),
    )(parts, b)


def kernel(raw_x, edge_index, W1, b1, W2, b2):
    src = edge_index[0].reshape(E // CHUNK, CHUNK)
    dst = edge_index[1].reshape(E // CHUNK, CHUNK)
    parts1 = _spmm_h(raw_x, src, dst)
    support2 = _fuse1(parts1, W1, b1.reshape(1, NHID), W2)
    parts2 = _spmm_c(support2, src, dst)
    return _lsm(parts2, b2.reshape(1, NCLASS))
